# baseline (speedup 1.0000x reference)
import jax
import jax.numpy as jnp
from jax import lax
from jax.experimental import pallas as pl
from jax.experimental.pallas import tpu as pltpu
from jax.experimental.pallas import tpu_sc as plsc

N = 16384
D = 256
L = 16
NC, NS = 2, 16
NW = NC * NS
CPW = D // NW
CH = 4
EPC = N // CH
VPC = EPC // L
RBITS = (11, 11, 10)
RSHIFT = (0, 11, 22)
HB = 1 << 11
IDX_BITS = 14
IDX_MASK = (1 << IDX_BITS) - 1
INV_N1 = 1.0 / (N + 1)


def _digits(k_i32, shift, mask):
  ku = plsc.bitcast(k_i32, jnp.uint32)
  return plsc.bitcast((ku >> shift) & mask, jnp.int32)


def _emit_pass(src_k, src_v, dst_k, dst_v, hist, pass_idx):
  shift = RSHIFT[pass_idx]
  mask = jnp.uint32((1 << RBITS[pass_idx]) - 1)
  nbins = 1 << RBITS[pass_idx]
  first = pass_idx == 0
  last = pass_idx == len(RBITS) - 1

  def zero_body(b, carry):
    hist[pl.ds(b * L, L)] = jnp.zeros((L,), jnp.int32)
    return carry

  lax.fori_loop(0, (CH * HB) // L, zero_body, 0, unroll=4)

  def hist_body(i, carry):
    for h in range(CH):
      sl = pl.ds(h * EPC + i * L, L)
      k = src_k[sl]
      if first:
        m = jnp.right_shift(k, 31)
        k = k ^ (m | jnp.int32(-(2 ** 31)))
        src_k[sl] = k
      if last:
        d = (k >> IDX_BITS) + (h * HB)
      else:
        d = _digits(k, shift, mask) + (h * HB)
      cnt, last_m = plsc.scan_count(d)
      base = plsc.load_gather(hist, [d])
      plsc.store_scatter(hist, [d], base + cnt, mask=last_m)
    return carry

  lax.fori_loop(0, VPC, hist_body, 0)

  def scan_body(b, carry):
    c = [hist[pl.ds(h * HB + b * L, L)] for h in range(CH)]
    t01 = c[0] + c[1]
    t = t01 + c[2] + c[3]
    s = plsc.cumsum(t)
    excl = s - t + carry
    hist[pl.ds(0 * HB + b * L, L)] = excl
    hist[pl.ds(1 * HB + b * L, L)] = excl + c[0]
    hist[pl.ds(2 * HB + b * L, L)] = excl + t01
    hist[pl.ds(3 * HB + b * L, L)] = excl + t01 + c[2]
    return carry + jnp.sum(t)

  lax.fori_loop(0, nbins // L, scan_body, jnp.int32(0))

  def perm_body(i, carry):
    for h in range(CH):
      sl = pl.ds(h * EPC + i * L, L)
      k = src_k[sl]
      if last:
        d = (k >> IDX_BITS) + (h * HB)
      else:
        d = _digits(k, shift, mask) + (h * HB)
      cnt, last_m = plsc.scan_count(d)
      base = plsc.load_gather(hist, [d])
      pos = base + cnt - 1
      if first:
        v = lax.iota(jnp.int32, L) + (h * EPC + i * L)
      else:
        v = src_v[sl] if src_v is not None else None
      if last:
        fbits = plsc.bitcast(
            (pos + 1).astype(jnp.float32) * INV_N1, jnp.int32)
        plsc.store_scatter(dst_k, [k & IDX_MASK], fbits)
      elif pass_idx == 1:
        ku = plsc.bitcast(k, jnp.uint32)
        packed = plsc.bitcast((ku >> 22) << IDX_BITS, jnp.int32) | v
        plsc.store_scatter(dst_k, [pos], packed)
      else:
        plsc.store_scatter(dst_k, [pos], k)
        plsc.store_scatter(dst_v, [pos], v)
      plsc.store_scatter(hist, [d], base + cnt, mask=last_m)
    return carry

  lax.fori_loop(0, VPC, perm_body, 0)


def _rank_body(st_hbm, out_hbm, k0, k1, v1, hist):
  wid = lax.axis_index("s") * NC + lax.axis_index("c")

  def col_body(ci, carry):
    col = wid * CPW + ci
    pltpu.sync_copy(st_hbm.at[col], k0)
    _emit_pass(k0, None, k1, v1, hist, 0)
    _emit_pass(k1, v1, k0, None, hist, 1)
    _emit_pass(k0, None, k1, None, hist, 2)
    pltpu.sync_copy(k1, out_hbm.at[col])
    return carry

  lax.fori_loop(0, CPW, col_body, 0)


@jax.jit
def _rank_columns(st_keys):
  mesh = plsc.VectorSubcoreMesh(
      core_axis_name="c", subcore_axis_name="s",
      num_cores=NC, num_subcores=NS)
  f = pl.kernel(
      _rank_body,
      out_type=jax.ShapeDtypeStruct((D, N), jnp.int32),
      mesh=mesh,
      compiler_params=pltpu.CompilerParams(needs_layout_passes=False),
      scratch_types=[
          pltpu.VMEM((N,), jnp.int32),
          pltpu.VMEM((N,), jnp.int32),
          pltpu.VMEM((N,), jnp.int32),
          pltpu.VMEM((CH * HB,), jnp.int32),
      ],
  )
  return f(st_keys)


def kernel(samples):
  st = jnp.transpose(samples).view(jnp.int32)
  ranks = _rank_columns(st)
  return ranks.view(jnp.float32)[None, :, :]


# speedup vs baseline: 2.4226x; 2.4226x over previous
import jax
import jax.numpy as jnp
from jax import lax
from jax.experimental import pallas as pl
from jax.experimental.pallas import tpu as pltpu
from jax.experimental.pallas import tpu_sc as plsc

N = 16384
D = 256
L = 16
NC, NS = 2, 16
NW = NC * NS
CPW = D // NW
CH = 4
EPC = N // CH
VPC = EPC // L
HB = EPC
CHAIN_MASK = -EPC
IDX_BITS = 14
IDX_MASK = (1 << IDX_BITS) - 1
INV_N1 = 1.0 / (N + 1)


def _transform(k):
  m = jnp.right_shift(k, 31)
  return k ^ (m | jnp.int32(-(2 ** 31)))


def _shr(k_i32, shift):
  ku = plsc.bitcast(k_i32, jnp.uint32)
  return plsc.bitcast(ku >> shift, jnp.int32)


def _zero(hist, nwords):
  def body(b, c):
    hist[pl.ds(b * L, L)] = jnp.zeros((L,), jnp.int32)
    return c
  lax.fori_loop(0, nwords // L, body, 0, unroll=8)


def _merge(hist, nbins):
  def body(b, carry):
    cs = [hist[pl.ds(h * HB + b * L, L)] for h in range(CH)]
    partials = [cs[0]]
    for h in range(1, CH):
      partials.append(partials[-1] + cs[h])
    t = partials[-1]
    s = plsc.cumsum(t)
    excl = s - t + carry
    hist[pl.ds(0 * HB + b * L, L)] = excl
    for h in range(1, CH):
      hist[pl.ds(h * HB + b * L, L)] = excl + partials[h - 1]
    return carry + jnp.sum(t)
  lax.fori_loop(0, nbins // L, body, jnp.int32(-1))


def _count_pre(hist, ds):
  scans = [plsc.scan_count(d) for d in ds]
  bases = [plsc.load_gather(hist, [d]) for d in ds]
  return scans, bases


def _count_commit(hist, ds, scans, bases):
  for h in range(CH):
    cnt, last_m = scans[h]
    plsc.store_scatter(hist, [ds[h]], bases[h] + cnt, mask=last_m)


def _histn(src, hist, extract, transform_in_place=False):
  def body(i, c):
    ks = [src[pl.ds(h * EPC + i * L, L)] for h in range(CH)]
    if transform_in_place:
      ks = [_transform(k) for k in ks]
    ds = [extract(ks[h]) + (h * HB) for h in range(CH)]
    scans, bases = _count_pre(hist, ds)
    if transform_in_place:
      for h in range(CH):
        src[pl.ds(h * EPC + i * L, L)] = ks[h]
    _count_commit(hist, ds, scans, bases)
    return c
  lax.fori_loop(0, VPC, body, 0)


def _d0(k):
  return k & 0x7FF


def _d1(k):
  return _shr(k, 11) & 0x7FF


def _d2(p):
  return _shr(p, IDX_BITS)


def _perm1(src, dst_k, dst_v, h_cur):
  def body(i, c):
    ks = [src[pl.ds(h * EPC + i * L, L)] for h in range(CH)]
    d0s = [(ks[h] & 0x7FF) + (h * HB) for h in range(CH)]
    scans0, bases0 = _count_pre(h_cur, d0s)
    poss = [bases0[h] + scans0[h][0] for h in range(CH)]
    _count_commit(h_cur, d0s, scans0, bases0)
    for h in range(CH):
      plsc.store_scatter(dst_k, [poss[h]], ks[h])
      v = lax.iota(jnp.int32, L) + (h * EPC + i * L)
      plsc.store_scatter(dst_v, [poss[h]], v)
    return c
  lax.fori_loop(0, VPC, body, 0)


def _perm2(src_k, src_v, dst_k, h_cur):
  def body(i, c):
    ks = [src_k[pl.ds(h * EPC + i * L, L)] for h in range(CH)]
    vs = [src_v[pl.ds(h * EPC + i * L, L)] for h in range(CH)]
    d1s = [(_shr(ks[h], 11) & 0x7FF) + (h * HB) for h in range(CH)]
    scans1, bases1 = _count_pre(h_cur, d1s)
    poss = [bases1[h] + scans1[h][0] for h in range(CH)]
    _count_commit(h_cur, d1s, scans1, bases1)
    tops = [_shr(ks[h], 22) for h in range(CH)]
    for h in range(CH):
      packed = plsc.bitcast(
          plsc.bitcast(tops[h], jnp.uint32) << IDX_BITS, jnp.int32) | vs[h]
      plsc.store_scatter(dst_k, [poss[h]], packed)
    return c
  lax.fori_loop(0, VPC, body, 0)


def _perm3(src_p, dst_f, h_cur):
  def body(i, c):
    ps = [src_p[pl.ds(h * EPC + i * L, L)] for h in range(CH)]
    d2s = [_shr(ps[h], IDX_BITS) + (h * HB) for h in range(CH)]
    scans2, bases2 = _count_pre(h_cur, d2s)
    poss = [bases2[h] + scans2[h][0] for h in range(CH)]
    _count_commit(h_cur, d2s, scans2, bases2)
    for h in range(CH):
      fbits = plsc.bitcast(
          (poss[h] + 1).astype(jnp.float32) * INV_N1, jnp.int32)
      plsc.store_scatter(dst_f, [ps[h] & IDX_MASK], fbits)
    return c
  lax.fori_loop(0, VPC, body, 0)


def _rank_body(st_hbm, out_hbm, ka, kb, k1, v1, ha, hb, sem_in, sem_out):
  wid = lax.axis_index("s") * NC + lax.axis_index("c")
  base = wid * CPW
  last = base + CPW - 1

  pltpu.async_copy(st_hbm.at[base], ka, sem_in)

  def pair_body(j, carry):
    for par in range(2):
      ci = 2 * j + par
      col = base + ci
      cur = ka if par == 0 else kb
      nxt = kb if par == 0 else ka
      pltpu.make_async_copy(st_hbm.at[col], cur, sem_in).wait()
      pltpu.async_copy(st_hbm.at[jnp.minimum(col + 1, last)], nxt, sem_in)
      _zero(ha, CH * HB)
      _histn(cur, ha, _d0, transform_in_place=True)
      _merge(ha, 2048)

      @pl.when(ci > 0)
      def _wait_prev_out():
        pltpu.make_async_copy(k1, out_hbm.at[col - 1], sem_out).wait()

      _perm1(cur, k1, v1, ha)
      _zero(hb, CH * HB)
      _histn(k1, hb, _d1)
      _merge(hb, 2048)
      _perm2(k1, v1, cur, hb)
      _zero(ha, CH * HB)
      _histn(cur, ha, _d2)
      _merge(ha, 1024)
      _perm3(cur, k1, ha)
      pltpu.async_copy(k1, out_hbm.at[col], sem_out)
    return carry

  lax.fori_loop(0, CPW // 2, pair_body, 0)
  pltpu.make_async_copy(st_hbm.at[last], ka, sem_in).wait()
  pltpu.make_async_copy(k1, out_hbm.at[last], sem_out).wait()


  @jax.jit
  def _rank_columns(st_keys):
    mesh = plsc.VectorSubcoreMesh(
        core_axis_name="c", subcore_axis_name="s",
        num_cores=NC, num_subcores=NS)
    f = pl.kernel(
        _rank_body,
        out_type=jax.ShapeDtypeStruct((D, N), jnp.int32),
        mesh=mesh,
        compiler_params=pltpu.CompilerParams(
            needs_layout_passes=False, use_tc_tiling_on_sc=False),
        scratch_types=[
            pltpu.VMEM((N,), jnp.int32),
            pltpu.VMEM((N,), jnp.int32),
            pltpu.VMEM((N,), jnp.int32),
            pltpu.VMEM((N,), jnp.int32),
            pltpu.VMEM((CH * HB,), jnp.int32),
            pltpu.VMEM((CH * HB,), jnp.int32),
            pltpu.SemaphoreType.DMA,
            pltpu.SemaphoreType.DMA,
        ],
    )
    return f(st_keys)


  def kernel(samples):
    st = jnp.transpose(samples).view(jnp.int32)
    ranks = _rank_columns(st)
    return ranks.view(jnp.float32)[None, :, :]
